# Initial kernel scaffold; baseline (speedup 1.0000x reference)
import jax
import jax.numpy as jnp
from jax.experimental import pallas as pl


def kernel(Z):
    raise NotImplementedError("write your pallas kernel here")


# speedup vs baseline: 4.6542x; 4.6542x over previous
import functools

import jax
import jax.numpy as jnp
from jax.experimental import pallas as pl
from jax.experimental.pallas import tpu as pltpu

N, L, C = 8, 576, 768
_INF = float("inf")


def _msm_kernel(zi_ref, zj_ref, out_ref, m1_ref, m2_ref):
    i = pl.program_id(0)
    j = pl.program_id(1)

    @pl.when(j == 0)
    def _init():
        m1_ref[...] = jnp.full((1, L), _INF, jnp.float32)
        m2_ref[...] = jnp.full((1, L), _INF, jnp.float32)

    @pl.when(i != j)
    def _update():
        zi = zi_ref[0]
        zj = zj_ref[0]
        h = jax.lax.dot_general(
            zj, zi, (((1,), (1,)), ((), ())),
            preferred_element_type=jnp.float32)
        b2 = jnp.sum(zj * zj, axis=1, keepdims=True)
        ones = jnp.ones((1, C), jnp.float32)
        a2 = jax.lax.dot_general(
            ones, zi * zi, (((1,), (1,)), ((), ())),
            preferred_element_type=jnp.float32)
        t = jnp.min(b2 - 2.0 * h, axis=0, keepdims=True)
        v = jnp.sqrt(jnp.maximum(a2 + t, 0.0))
        m1 = m1_ref[...]
        m2 = m2_ref[...]
        m1_ref[...] = jnp.minimum(m1, v)
        m2_ref[...] = jnp.minimum(m2, jnp.maximum(m1, v))

    @pl.when(j == N - 1)
    def _finish():
        out_ref[0] = 0.5 * (m1_ref[...] + m2_ref[...])


@jax.jit
def kernel(Z):
    grid = (N, N)
    out = pl.pallas_call(
        _msm_kernel,
        grid=grid,
        in_specs=[
            pl.BlockSpec((1, L, C), lambda i, j: (i, 0, 0)),
            pl.BlockSpec((1, L, C), lambda i, j: (j, 0, 0)),
        ],
        out_specs=pl.BlockSpec((1, 1, L), lambda i, j: (i, 0, 0)),
        out_shape=jax.ShapeDtypeStruct((N, 1, L), jnp.float32),
        scratch_shapes=[
            pltpu.VMEM((1, L), jnp.float32),
            pltpu.VMEM((1, L), jnp.float32),
        ],
    )(Z, Z)
    return out[:, 0, :]
